# Initial kernel scaffold; baseline (speedup 1.0000x reference)
import jax
import jax.numpy as jnp
from jax.experimental import pallas as pl


def kernel(x, lens, W, label_seqs, sos):
    raise NotImplementedError("write your pallas kernel here")


# speedup vs baseline: 27.9825x; 27.9825x over previous
import functools

import jax
import jax.numpy as jnp
from jax.experimental import pallas as pl
from jax.experimental.pallas import tpu as pltpu


def _greedy_search_kernel(L_ref, W_ref, sos_ref, c_ref, q_ref, *, C, T_l, J):
    W = W_ref[:]
    L = L_ref[:]

    cls_iota = jax.lax.broadcasted_iota(jnp.int32, (C, 1), 0)
    s_iota = jax.lax.broadcasted_iota(jnp.int32, (1, T_l), 1)

    def argmin_col(sim):
        m = jnp.min(sim)
        idxs = jnp.where(sim == m, cls_iota, C)
        return jnp.min(idxs)

    q0 = jnp.tanh(jnp.dot(sos_ref[:], W, preferred_element_type=jnp.float32))
    d0 = jnp.sum((L_ref[:, 0, :] - q0) ** 2, axis=-1, keepdims=True)
    c = argmin_col(d0)

    def body(t, c):
        chosen = L_ref[pl.ds(c, 1), :, :].reshape(T_l, J)
        q = jnp.tanh(jnp.dot(chosen, W, preferred_element_type=jnp.float32))
        q_ref[:] = q
        d = jnp.sum((L - q[None, :, :]) ** 2, axis=-1)
        mask = (s_iota < t).astype(jnp.float32)
        sim = jnp.sum(d * mask, axis=-1, keepdims=True)
        return argmin_col(sim)

    c = jax.lax.fori_loop(1, T_l + 1, body, c)
    c_ref[:] = jnp.full((8, 128), c, dtype=jnp.int32)


def kernel(x, lens, W, label_seqs, sos):
    B = x.shape[0]
    C, T_l, J = label_seqs.shape

    c_tile, q = pl.pallas_call(
        functools.partial(_greedy_search_kernel, C=C, T_l=T_l, J=J),
        out_shape=(
            jax.ShapeDtypeStruct((8, 128), jnp.int32),
            jax.ShapeDtypeStruct((T_l, J), jnp.float32),
        ),
    )(label_seqs, W, sos.reshape(1, J))

    pred_label_sofar = jnp.broadcast_to(c_tile[0, 0], (B,))
    pred_label_seq = jnp.broadcast_to(q[None, :, :], (B, T_l, J))
    return (pred_label_sofar, pred_label_seq)
